# baseline (speedup 1.0000x reference)
import functools

import jax
import jax.numpy as jnp
from jax import lax
from jax.experimental import pallas as pl
from jax.experimental.pallas import tpu as pltpu
from jax.experimental.pallas import tpu_sc as plsc

N_CHARS = 65536
N = 8388608

_NC = 2
_NS = 16
_NW = _NC * _NS
_L = 16

_PER_W = N // _NW
_CHUNK = 4096
_NUM_CHUNKS = _PER_W // _CHUNK


def _log_table_body(counts_ref, out_ref):
    c = counts_ref[...]
    s = jnp.sum(c)
    out_ref[...] = jnp.log(c) - jnp.log(s)


def _compute_log_table(counts):
    c2 = counts.reshape(N_CHARS // 128, 128)
    out = pl.pallas_call(
        _log_table_body,
        out_shape=jax.ShapeDtypeStruct((N_CHARS // 128, 128), jnp.float32),
    )(c2)
    return out.reshape(N_CHARS)


_NBUF = 4
_BATCH = 8


@functools.partial(
    pl.kernel,
    mesh=plsc.VectorSubcoreMesh(core_axis_name="c", subcore_axis_name="s"),
    out_type=jax.ShapeDtypeStruct((N,), jnp.float32),
    compiler_params=pltpu.CompilerParams(
        needs_layout_passes=False,
        skip_device_barrier=True,
        disable_bounds_checks=True,
        disable_semaphore_checks=True,
    ),
    scratch_types=[
        pltpu.VMEM((N_CHARS,), jnp.float32),
        pltpu.VMEM_SHARED((N_CHARS,), jnp.float32),
    ] + [pltpu.VMEM((_CHUNK,), jnp.int32)] * _NBUF
      + [pltpu.VMEM((_CHUNK,), jnp.float32)] * _NBUF
      + [pltpu.SemaphoreType.DMA] * (2 * _NBUF),
)
def _sc_gather(table_hbm, idx_hbm, out_hbm, table_v, table_s, *bufs):
    idx_b = bufs[:_NBUF]
    out_b = bufs[_NBUF:2 * _NBUF]
    sin_b = bufs[2 * _NBUF:3 * _NBUF]
    sout_b = bufs[3 * _NBUF:4 * _NBUF]

    wid = lax.axis_index("s") * _NC + lax.axis_index("c")
    base = wid * _PER_W

    def in_slice(cur):
        off = pl.multiple_of(base + cur * _CHUNK, _CHUNK)
        return idx_hbm.at[pl.ds(off, _CHUNK)]

    def out_slice(cur):
        off = pl.multiple_of(base + cur * _CHUNK, _CHUNK)
        return out_hbm.at[pl.ds(off, _CHUNK)]

    for b in range(_NBUF):
        pltpu.async_copy(in_slice(b), idx_b[b], sin_b[b])

    @pl.when(lax.axis_index("s") == 0)
    def _():
        pltpu.sync_copy(table_hbm, table_s)
    plsc.subcore_barrier()
    pltpu.sync_copy(table_s, table_v)

    @pl.loop(0, _NUM_CHUNKS, step=_NBUF)
    def outer(ci):
        for b in range(_NBUF):
            cur = ci + b
            pltpu.make_async_copy(in_slice(cur), idx_b[b], sin_b[b]).wait()
            @pl.when(cur >= _NBUF)
            def _():
                pltpu.make_async_copy(
                    out_b[b], out_slice(cur - _NBUF), sout_b[b]).wait()

            def inner(j, _):
                o = pl.multiple_of(j * (_L * _BATCH), _L * _BATCH)
                ivs = [idx_b[b][pl.ds(o + k * _L, _L)] for k in range(_BATCH)]
                vals = [plsc.load_gather(table_v, [iv]) for iv in ivs]
                for k in range(_BATCH):
                    out_b[b][pl.ds(o + k * _L, _L)] = vals[k]
                return 0

            lax.fori_loop(0, _CHUNK // (_L * _BATCH), inner, 0)
            pltpu.async_copy(out_b[b], out_slice(cur), sout_b[b])

            @pl.when(cur + _NBUF < _NUM_CHUNKS)
            def _():
                pltpu.async_copy(in_slice(cur + _NBUF), idx_b[b], sin_b[b])

    for b in range(_NBUF):
        pltpu.make_async_copy(
            out_b[b], out_slice(_NUM_CHUNKS - _NBUF + b), sout_b[b]).wait()


def kernel(chars, counts):
    log_table = _compute_log_table(counts)
    return _sc_gather(log_table, chars)


# speedup vs baseline: 1.0101x; 1.0101x over previous
import functools

import jax
import jax.numpy as jnp
from jax import lax
from jax.experimental import pallas as pl
from jax.experimental.pallas import tpu as pltpu
from jax.experimental.pallas import tpu_sc as plsc

N_CHARS = 65536
N = 8388608

_NC = 2
_NS = 16
_NW = _NC * _NS
_L = 16

_PER_W = N // _NW
_CHUNK = 2048
_NUM_CHUNKS = _PER_W // _CHUNK


def _log_table_body(counts_ref, out_ref):
    c = counts_ref[...]
    s = jnp.sum(c)
    out_ref[...] = jnp.log(c) - jnp.log(s)


def _compute_log_table(counts):
    c2 = counts.reshape(N_CHARS // 128, 128)
    out = pl.pallas_call(
        _log_table_body,
        out_shape=jax.ShapeDtypeStruct((N_CHARS // 128, 128), jnp.float32),
    )(c2)
    return out.reshape(N_CHARS)


_NBUF = 8
_BATCH = 8


@functools.partial(
    pl.kernel,
    mesh=plsc.VectorSubcoreMesh(core_axis_name="c", subcore_axis_name="s"),
    out_type=jax.ShapeDtypeStruct((N,), jnp.float32),
    compiler_params=pltpu.CompilerParams(needs_layout_passes=False),
    scratch_types=[
        pltpu.VMEM((N_CHARS,), jnp.float32),
        pltpu.VMEM_SHARED((N_CHARS,), jnp.float32),
    ] + [pltpu.VMEM((_CHUNK,), jnp.int32)] * _NBUF
      + [pltpu.VMEM((_CHUNK,), jnp.float32)] * _NBUF
      + [pltpu.SemaphoreType.DMA] * (2 * _NBUF),
)
def _sc_gather(table_hbm, idx_hbm, out_hbm, table_v, table_s, *bufs):
    idx_b = bufs[:_NBUF]
    out_b = bufs[_NBUF:2 * _NBUF]
    sin_b = bufs[2 * _NBUF:3 * _NBUF]
    sout_b = bufs[3 * _NBUF:4 * _NBUF]

    wid = lax.axis_index("s") * _NC + lax.axis_index("c")
    base = wid * _PER_W

    def in_slice(cur):
        off = pl.multiple_of(base + cur * _CHUNK, _CHUNK)
        return idx_hbm.at[pl.ds(off, _CHUNK)]

    def out_slice(cur):
        off = pl.multiple_of(base + cur * _CHUNK, _CHUNK)
        return out_hbm.at[pl.ds(off, _CHUNK)]

    for b in range(_NBUF):
        pltpu.async_copy(in_slice(b), idx_b[b], sin_b[b])

    @pl.when(lax.axis_index("s") == 0)
    def _():
        pltpu.sync_copy(table_hbm, table_s)
    plsc.subcore_barrier()
    pltpu.sync_copy(table_s, table_v)

    @pl.loop(0, _NUM_CHUNKS, step=_NBUF)
    def outer(ci):
        for b in range(_NBUF):
            cur = ci + b
            pltpu.make_async_copy(in_slice(cur), idx_b[b], sin_b[b]).wait()
            @pl.when(cur >= _NBUF)
            def _():
                pltpu.make_async_copy(
                    out_b[b], out_slice(cur - _NBUF), sout_b[b]).wait()

            def inner(j, _):
                o = pl.multiple_of(j * (_L * _BATCH), _L * _BATCH)
                ivs = [idx_b[b][pl.ds(o + k * _L, _L)] for k in range(_BATCH)]
                vals = [plsc.load_gather(table_v, [iv]) for iv in ivs]
                for k in range(_BATCH):
                    out_b[b][pl.ds(o + k * _L, _L)] = vals[k]
                return 0

            lax.fori_loop(0, _CHUNK // (_L * _BATCH), inner, 0)
            pltpu.async_copy(out_b[b], out_slice(cur), sout_b[b])

            @pl.when(cur + _NBUF < _NUM_CHUNKS)
            def _():
                pltpu.async_copy(in_slice(cur + _NBUF), idx_b[b], sin_b[b])

    for b in range(_NBUF):
        pltpu.make_async_copy(
            out_b[b], out_slice(_NUM_CHUNKS - _NBUF + b), sout_b[b]).wait()


def kernel(chars, counts):
    log_table = _compute_log_table(counts)
    return _sc_gather(log_table, chars)


# speedup vs baseline: 1.0114x; 1.0012x over previous
import functools

import jax
import jax.numpy as jnp
from jax import lax
from jax.experimental import pallas as pl
from jax.experimental.pallas import tpu as pltpu
from jax.experimental.pallas import tpu_sc as plsc

N_CHARS = 65536
N = 8388608

_NC = 2
_NS = 16
_NW = _NC * _NS
_L = 16

_PER_W = N // _NW
_CHUNK = 2048
_NUM_CHUNKS = _PER_W // _CHUNK


def _log_table_body(counts_ref, out_ref):
    c = counts_ref[...]
    s = jnp.sum(c)
    out_ref[...] = jnp.log(c) - jnp.log(s)


def _compute_log_table(counts):
    c2 = counts.reshape(N_CHARS // 128, 128)
    out = pl.pallas_call(
        _log_table_body,
        out_shape=jax.ShapeDtypeStruct((N_CHARS // 128, 128), jnp.float32),
    )(c2)
    return out.reshape(N_CHARS)


_NBUF = 8
_BATCH = 8


@functools.partial(
    pl.kernel,
    mesh=plsc.VectorSubcoreMesh(core_axis_name="c", subcore_axis_name="s"),
    out_type=jax.ShapeDtypeStruct((N,), jnp.float32),
    compiler_params=pltpu.CompilerParams(needs_layout_passes=False),
    scratch_types=[
        pltpu.VMEM((N_CHARS,), jnp.float32),
        pltpu.VMEM_SHARED((N_CHARS,), jnp.float32),
    ] + [pltpu.VMEM((_CHUNK,), jnp.int32)] * _NBUF
      + [pltpu.VMEM((_CHUNK,), jnp.float32)] * _NBUF
      + [pltpu.SemaphoreType.DMA] * (2 * _NBUF),
)
def _sc_gather(table_hbm, idx_hbm, out_hbm, table_v, table_s, *bufs):
    idx_b = bufs[:_NBUF]
    out_b = bufs[_NBUF:2 * _NBUF]
    sin_b = bufs[2 * _NBUF:3 * _NBUF]
    sout_b = bufs[3 * _NBUF:4 * _NBUF]

    wid = lax.axis_index("s") * _NC + lax.axis_index("c")
    base = wid * _PER_W

    def in_slice(cur):
        off = pl.multiple_of(base + cur * _CHUNK, _CHUNK)
        return idx_hbm.at[pl.ds(off, _CHUNK)]

    def out_slice(cur):
        off = pl.multiple_of(base + cur * _CHUNK, _CHUNK)
        return out_hbm.at[pl.ds(off, _CHUNK)]

    for b in range(_NBUF):
        pltpu.async_copy(in_slice(b), idx_b[b], sin_b[b])

    _SLICE = N_CHARS // _NS
    soff = pl.multiple_of(lax.axis_index("s") * _SLICE, _SLICE)
    pltpu.sync_copy(table_hbm.at[pl.ds(soff, _SLICE)],
                    table_s.at[pl.ds(soff, _SLICE)])
    plsc.subcore_barrier()
    pltpu.sync_copy(table_s, table_v)

    @pl.loop(0, _NUM_CHUNKS, step=_NBUF)
    def outer(ci):
        for b in range(_NBUF):
            cur = ci + b
            pltpu.make_async_copy(in_slice(cur), idx_b[b], sin_b[b]).wait()
            @pl.when(cur >= _NBUF)
            def _():
                pltpu.make_async_copy(
                    out_b[b], out_slice(cur - _NBUF), sout_b[b]).wait()

            def inner(j, _):
                o = pl.multiple_of(j * (_L * _BATCH), _L * _BATCH)
                ivs = [idx_b[b][pl.ds(o + k * _L, _L)] for k in range(_BATCH)]
                vals = [plsc.load_gather(table_v, [iv]) for iv in ivs]
                for k in range(_BATCH):
                    out_b[b][pl.ds(o + k * _L, _L)] = vals[k]
                return 0

            lax.fori_loop(0, _CHUNK // (_L * _BATCH), inner, 0)
            pltpu.async_copy(out_b[b], out_slice(cur), sout_b[b])

            @pl.when(cur + _NBUF < _NUM_CHUNKS)
            def _():
                pltpu.async_copy(in_slice(cur + _NBUF), idx_b[b], sin_b[b])

    for b in range(_NBUF):
        pltpu.make_async_copy(
            out_b[b], out_slice(_NUM_CHUNKS - _NBUF + b), sout_b[b]).wait()


def kernel(chars, counts):
    log_table = _compute_log_table(counts)
    return _sc_gather(log_table, chars)
